# speedup vs baseline: 153.4856x; 153.4856x over previous
import functools

import jax
import jax.numpy as jnp
from jax.experimental import pallas as pl
from jax.experimental.pallas import tpu as pltpu

B, N, C = 64, 576, 768
K_TOTAL = int(B * N * C * 0.1)
KP = -(-K_TOTAL // B)
N_BLK = 64


def _score_rank_kernel(w_e_ref, w_o_ref, mt_ref, b_e_ref, b_o_ref, nk_ref):
    t0 = jnp.dot(w_e_ref[...], mt_ref[...], preferred_element_type=jnp.float32) + b_e_ref[...]
    t1 = jnp.dot(w_o_ref[...], mt_ref[...], preferred_element_type=jnp.float32) + b_o_ref[...]
    mx = jnp.maximum(t0, t1)
    s0 = t0 - mx
    s1 = t1 - mx
    lse = jnp.log(jnp.exp(s0) + jnp.exp(s1))
    lp0 = s0 - lse
    lp1 = s1 - lse
    mx2 = jnp.maximum(lp0, lp1)
    e0 = jnp.exp(lp0 - mx2)
    e1 = jnp.exp(lp1 - mx2)
    s = e0 / (e0 + e1)

    u = jax.lax.bitcast_convert_type(s, jnp.int32)

    def body(_, carry):
        lo, hi = carry
        mid = lo + (hi - lo) // 2
        cnt = jnp.sum((u >= mid).astype(jnp.int32))
        big = cnt >= KP
        return (jnp.where(big, mid, lo), jnp.where(big, hi, mid))

    lo, _ = jax.lax.fori_loop(0, 31, body, (jnp.int32(0), jnp.int32(0x7F800000)))
    v = lo
    g = jnp.sum((u > v).astype(jnp.int32))
    m = jnp.sum((u == v).astype(jnp.int32))
    r = jnp.float32(K_TOTAL) - 64.0 * g.astype(jnp.float32)

    tied = (u == v).astype(jnp.float32)
    ic0 = jax.lax.broadcasted_iota(jnp.int32, (C, C), 0)
    ic1 = jax.lax.broadcasted_iota(jnp.int32, (C, C), 1)
    upper = (ic0 < ic1).astype(jnp.float32)
    within = jnp.dot(tied, upper, preferred_element_type=jnp.float32)
    in0 = jax.lax.broadcasted_iota(jnp.int32, (N, N), 0)
    in1 = jax.lax.broadcasted_iota(jnp.int32, (N, N), 1)
    lower = (in1 < in0).astype(jnp.float32)
    row_tot = jnp.sum(tied, axis=1, keepdims=True)
    cross = jnp.dot(lower, row_tot, preferred_element_type=jnp.float32)
    j = within + cross

    mf = m.astype(jnp.float32)
    n_tied = jnp.clip(jnp.floor((r - j + mf - 1.0) / mf), 0.0, 64.0)
    nk = jnp.where(u > v, jnp.float32(B), jnp.where(u == v, n_tied, 0.0))
    nk_ref[...] = nk.astype(jnp.int32)


def _mask_ln_kernel(x_ref, tok_ref, nk_ref, g_ref, b_ref, out_ref, mask_ref):
    b = pl.program_id(1)
    keep = nk_ref[...] > b
    keep3 = keep[None, :, :]
    xm = jnp.where(keep3, x_ref[...], tok_ref[...])
    mask_ref[...] = keep3.astype(jnp.float32)
    mu = jnp.mean(xm, axis=-1, keepdims=True)
    d = xm - mu
    var = jnp.mean(d * d, axis=-1, keepdims=True)
    out_ref[...] = d / jnp.sqrt(var + 1e-5) * g_ref[...] + b_ref[...]


def kernel(x, patch_mask_para, fc_W, fc_b, learnable_token, ln_gamma, ln_beta):
    mt = patch_mask_para.transpose(0, 2, 1).reshape(2 * N, C)
    w_e = fc_W[0::2, :]
    w_o = fc_W[1::2, :]
    b_e = fc_b[0::2][:, None]
    b_o = fc_b[1::2][:, None]

    n_keep = pl.pallas_call(
        _score_rank_kernel,
        out_shape=jax.ShapeDtypeStruct((N, C), jnp.int32),
    )(w_e, w_o, mt, b_e, b_o)

    grid = (N // N_BLK, B)
    out, mask = pl.pallas_call(
        _mask_ln_kernel,
        grid=grid,
        in_specs=[
            pl.BlockSpec((1, N_BLK, C), lambda j, b: (b, j, 0)),
            pl.BlockSpec((1, N_BLK, C), lambda j, b: (0, j, 0)),
            pl.BlockSpec((N_BLK, C), lambda j, b: (j, 0)),
            pl.BlockSpec((1, C), lambda j, b: (0, 0)),
            pl.BlockSpec((1, C), lambda j, b: (0, 0)),
        ],
        out_specs=[
            pl.BlockSpec((1, N_BLK, C), lambda j, b: (b, j, 0)),
            pl.BlockSpec((1, N_BLK, C), lambda j, b: (b, j, 0)),
        ],
        out_shape=[
            jax.ShapeDtypeStruct((B, N, C), jnp.float32),
            jax.ShapeDtypeStruct((B, N, C), jnp.float32),
        ],
    )(x, learnable_token, n_keep, ln_gamma[None, :], ln_beta[None, :])
    return out, mask


# speedup vs baseline: 393.5548x; 2.5641x over previous
import functools

import jax
import jax.numpy as jnp
from jax.experimental import pallas as pl
from jax.experimental.pallas import tpu as pltpu

B, N, C = 64, 576, 768
K_TOTAL = int(B * N * C * 0.1)
KP = -(-K_TOTAL // B)
N_BLK = 64


def _score_rank_kernel(w_e_ref, w_o_ref, mt_ref, b_e_ref, b_o_ref, nk_ref):
    t0 = jnp.dot(w_e_ref[...], mt_ref[...], preferred_element_type=jnp.float32) + b_e_ref[...]
    t1 = jnp.dot(w_o_ref[...], mt_ref[...], preferred_element_type=jnp.float32) + b_o_ref[...]
    mx = jnp.maximum(t0, t1)
    s0 = t0 - mx
    s1 = t1 - mx
    lse = jnp.log(jnp.exp(s0) + jnp.exp(s1))
    lp0 = s0 - lse
    lp1 = s1 - lse
    mx2 = jnp.maximum(lp0, lp1)
    e0 = jnp.exp(lp0 - mx2)
    e1 = jnp.exp(lp1 - mx2)
    s = e0 / (e0 + e1)

    u = jax.lax.bitcast_convert_type(s, jnp.int32)

    def body(_, carry):
        lo, hi = carry
        mid = lo + (hi - lo) // 2
        cnt = jnp.sum((u >= mid).astype(jnp.int32))
        big = cnt >= KP
        return (jnp.where(big, mid, lo), jnp.where(big, hi, mid))

    lo, _ = jax.lax.fori_loop(0, 31, body, (jnp.int32(0), jnp.int32(0x7F800000)))
    v = lo
    g = jnp.sum((u > v).astype(jnp.int32))
    m = jnp.sum((u == v).astype(jnp.int32))
    r = jnp.float32(K_TOTAL) - 64.0 * g.astype(jnp.float32)

    tied = (u == v).astype(jnp.float32)
    ic0 = jax.lax.broadcasted_iota(jnp.int32, (C, C), 0)
    ic1 = jax.lax.broadcasted_iota(jnp.int32, (C, C), 1)
    upper = (ic0 < ic1).astype(jnp.float32)
    within = jnp.dot(tied, upper, preferred_element_type=jnp.float32)
    in0 = jax.lax.broadcasted_iota(jnp.int32, (N, N), 0)
    in1 = jax.lax.broadcasted_iota(jnp.int32, (N, N), 1)
    lower = (in1 < in0).astype(jnp.float32)
    row_tot = jnp.sum(tied, axis=1, keepdims=True)
    cross = jnp.dot(lower, row_tot, preferred_element_type=jnp.float32)
    j = within + cross

    mf = m.astype(jnp.float32)
    n_tied = jnp.clip(jnp.floor((r - j + mf - 1.0) / mf), 0.0, 64.0)
    nk = jnp.where(u > v, jnp.float32(B), jnp.where(u == v, n_tied, 0.0))
    nk_ref[...] = nk.astype(jnp.int32)


def _mask_ln_kernel(x_ref, tok_ref, nk_ref, g_ref, b_ref, out_ref, mask_ref):
    b = pl.program_id(0)
    keep = nk_ref[...] > b
    keep3 = keep[None, :, :]
    xm = jnp.where(keep3, x_ref[...], tok_ref[...])
    mask_ref[...] = keep3.astype(jnp.float32)
    mu = jnp.mean(xm, axis=-1, keepdims=True)
    d = xm - mu
    var = jnp.mean(d * d, axis=-1, keepdims=True)
    out_ref[...] = d / jnp.sqrt(var + 1e-5) * g_ref[...] + b_ref[...]


def kernel(x, patch_mask_para, fc_W, fc_b, learnable_token, ln_gamma, ln_beta):
    mt = patch_mask_para.transpose(0, 2, 1).reshape(2 * N, C)
    w_e = fc_W[0::2, :]
    w_o = fc_W[1::2, :]
    b_e = fc_b[0::2][:, None]
    b_o = fc_b[1::2][:, None]

    n_keep = pl.pallas_call(
        _score_rank_kernel,
        out_shape=jax.ShapeDtypeStruct((N, C), jnp.int32),
    )(w_e, w_o, mt, b_e, b_o)

    grid = (B,)
    out, mask = pl.pallas_call(
        _mask_ln_kernel,
        grid=grid,
        in_specs=[
            pl.BlockSpec((1, N, C), lambda b: (b, 0, 0)),
            pl.BlockSpec((1, N, C), lambda b: (0, 0, 0)),
            pl.BlockSpec((N, C), lambda b: (0, 0)),
            pl.BlockSpec((1, C), lambda b: (0, 0)),
            pl.BlockSpec((1, C), lambda b: (0, 0)),
        ],
        out_specs=[
            pl.BlockSpec((1, N, C), lambda b: (b, 0, 0)),
            pl.BlockSpec((1, N, C), lambda b: (b, 0, 0)),
        ],
        out_shape=[
            jax.ShapeDtypeStruct((B, N, C), jnp.float32),
            jax.ShapeDtypeStruct((B, N, C), jnp.float32),
        ],
    )(x, learnable_token, n_keep, ln_gamma[None, :], ln_beta[None, :])
    return out, mask
